# Initial kernel scaffold; baseline (speedup 1.0000x reference)
import jax
import jax.numpy as jnp
from jax.experimental import pallas as pl


def kernel(atomic_numbers, pos, edge_index, batch, emb_table, W1, b1, W2, b2, W_out):
    raise NotImplementedError("write your pallas kernel here")


# speedup vs baseline: 6.3048x; 6.3048x over previous
import functools

import jax
import jax.numpy as jnp
from jax import lax
from jax.experimental import pallas as pl
from jax.experimental.pallas import tpu as pltpu
from jax.experimental.pallas import tpu_sc as plsc

_N_NODES = 10000
_N_EDGES = 320000
_N_GRAPHS = 64
_C = 128
_NUM_GAUSS = 600
_CUTOFF = 5.0
_AVG_DEGREE = 32.0
_AVG_NUM_NODES = 156.25

_NW = 32
_EPW = _N_EDGES // _NW
_A_CH = 2000
_A_NCH = _EPW // _A_CH
_S_CH = 200
_S_NCH = _EPW // _S_CH
_ROWS_PER_TILE = _N_NODES // 16

_TE = 2000
_NT = _N_EDGES // _TE
_EMB_PAD = 96

_DELTA = float(_CUTOFF) / (_NUM_GAUSS - 1)
_COEFF = -0.5 / (2.0 * _DELTA) ** 2


def _sc_geom_body(posx_hbm, posy_hbm, posz_hbm, src_hbm, dst_hbm, an_hbm,
                  d2_hbm, ansrc_hbm,
                  posx_v, posy_v, posz_v, an_v, sidx_v, didx_v, d2_v, ansrc_v):
    cid = lax.axis_index("c")
    sid = lax.axis_index("s")
    wid = sid * 2 + cid
    base = wid * _EPW
    pltpu.sync_copy(posx_hbm, posx_v)
    pltpu.sync_copy(posy_hbm, posy_v)
    pltpu.sync_copy(posz_hbm, posz_v)
    pltpu.sync_copy(an_hbm, an_v)

    def chunk(c, carry):
        cb = base + c * _A_CH
        pltpu.sync_copy(src_hbm.at[pl.ds(cb, _A_CH)], sidx_v)
        pltpu.sync_copy(dst_hbm.at[pl.ds(cb, _A_CH)], didx_v)

        def body(i, carry2):
            s = sidx_v[pl.ds(i * 16, 16)]
            d = didx_v[pl.ds(i * 16, 16)]
            dx = plsc.load_gather(posx_v, [s]) - plsc.load_gather(posx_v, [d])
            dy = plsc.load_gather(posy_v, [s]) - plsc.load_gather(posy_v, [d])
            dz = plsc.load_gather(posz_v, [s]) - plsc.load_gather(posz_v, [d])
            d2_v[pl.ds(i * 16, 16)] = dx * dx + dy * dy + dz * dz
            ansrc_v[pl.ds(i * 16, 16)] = plsc.load_gather(an_v, [s])
            return carry2

        lax.fori_loop(0, _A_CH // 16, body, 0)
        pltpu.sync_copy(d2_v, d2_hbm.at[pl.ds(cb, _A_CH)])
        pltpu.sync_copy(ansrc_v, ansrc_hbm.at[pl.ds(cb, _A_CH)])
        return carry

    lax.fori_loop(0, _A_NCH, chunk, 0)


def _sc_geom(posx, posy, posz, src, dst, an):
    mesh = plsc.VectorSubcoreMesh(core_axis_name="c", subcore_axis_name="s")
    fn = pl.kernel(
        _sc_geom_body,
        mesh=mesh,
        out_type=[
            jax.ShapeDtypeStruct((_N_EDGES,), jnp.float32),
            jax.ShapeDtypeStruct((_N_EDGES,), jnp.int32),
        ],
        scratch_types=[
            pltpu.VMEM((_N_NODES,), jnp.float32),
            pltpu.VMEM((_N_NODES,), jnp.float32),
            pltpu.VMEM((_N_NODES,), jnp.float32),
            pltpu.VMEM((_N_NODES,), jnp.int32),
            pltpu.VMEM((_A_CH,), jnp.int32),
            pltpu.VMEM((_A_CH,), jnp.int32),
            pltpu.VMEM((_A_CH,), jnp.float32),
            pltpu.VMEM((_A_CH,), jnp.int32),
        ],
        compiler_params=pltpu.CompilerParams(needs_layout_passes=False),
    )
    return fn(posx, posy, posz, src, dst, an)


def _tc_edge_body(d2_ref, ansrc_ref, emb_ref, w1_ref, b1_ref, w2_ref, b2_ref,
                  out_ref):
    d2 = d2_ref[0]
    dist = jnp.sqrt(d2 + 1e-8)
    offs = lax.broadcasted_iota(
        jnp.int32, (_NUM_GAUSS, _TE), 0).astype(jnp.float32) * _DELTA
    dm = dist - offs
    rbf_t = jnp.exp(_COEFF * dm * dm)
    h = lax.dot_general(rbf_t, w1_ref[...], (((0,), (0,)), ((), ())),
                        preferred_element_type=jnp.float32)
    h = jax.nn.silu(h + b1_ref[...])
    ef = jax.nn.silu(
        jnp.dot(h, w2_ref[...], preferred_element_type=jnp.float32)
        + b2_ref[...])
    an_row = ansrc_ref[0]
    ids = lax.broadcasted_iota(jnp.int32, (_EMB_PAD, _TE), 0)
    onehot_t = (ids == an_row).astype(jnp.float32)
    xsrc = lax.dot_general(onehot_t, emb_ref[...], (((0,), (0,)), ((), ())),
                           preferred_element_type=jnp.float32)
    out_ref[...] = ef * xsrc


def _tc_edge(d2, ansrc, emb_pad, W1, b1, W2, b2):
    d2_3d = d2.reshape(_NT, 1, _TE)
    an_3d = ansrc.reshape(_NT, 1, _TE)
    return pl.pallas_call(
        _tc_edge_body,
        grid=(_NT,),
        in_specs=[
            pl.BlockSpec((1, 1, _TE), lambda i: (i, 0, 0)),
            pl.BlockSpec((1, 1, _TE), lambda i: (i, 0, 0)),
            pl.BlockSpec((_EMB_PAD, _C), lambda i: (0, 0)),
            pl.BlockSpec((_NUM_GAUSS, _C), lambda i: (0, 0)),
            pl.BlockSpec((1, _C), lambda i: (0, 0)),
            pl.BlockSpec((_C, _C), lambda i: (0, 0)),
            pl.BlockSpec((1, _C), lambda i: (0, 0)),
        ],
        out_specs=pl.BlockSpec((_TE, _C), lambda i: (i, 0)),
        out_shape=jax.ShapeDtypeStruct((_N_EDGES, _C), jnp.float32),
        compiler_params=pltpu.CompilerParams(
            dimension_semantics=("arbitrary",)),
    )(d2_3d, an_3d, emb_pad, W1, b1.reshape(1, _C), W2, b2.reshape(1, _C))


def _sc_scatter_body(msg_hbm, dst_hbm, zeros_hbm, agg_hbm,
                     msg_v, idx_v, agg_sp):
    cid = lax.axis_index("c")
    sid = lax.axis_index("s")
    wid = sid * 2 + cid
    base = wid * _EPW

    @pl.when(sid == 0)
    def _():
        pltpu.sync_copy(zeros_hbm, agg_sp)

    plsc.subcore_barrier()

    def chunk(c, carry):
        cb = base + c * _S_CH
        pltpu.sync_copy(dst_hbm.at[pl.ds(cb, _S_CH)], idx_v)
        pltpu.sync_copy(msg_hbm.at[pl.ds(cb, _S_CH)], msg_v)
        pltpu.sync_copy(msg_v, agg_sp.at[idx_v], add=True)
        return carry

    lax.fori_loop(0, _S_NCH, chunk, 0)
    plsc.subcore_barrier()
    rb = sid * 624
    pltpu.sync_copy(agg_sp.at[pl.ds(rb, 624)],
                    agg_hbm.at[cid, pl.ds(rb, 624)])

    @pl.when(sid == 15)
    def _():
        pltpu.sync_copy(agg_sp.at[pl.ds(9984, 16)],
                        agg_hbm.at[cid, pl.ds(9984, 16)])


def _sc_scatter(msg, dst, zeros):
    mesh = plsc.VectorSubcoreMesh(core_axis_name="c", subcore_axis_name="s")
    fn = pl.kernel(
        _sc_scatter_body,
        mesh=mesh,
        out_type=jax.ShapeDtypeStruct((2, _N_NODES, _C), jnp.float32),
        scratch_types=[
            pltpu.VMEM((_S_CH, _C), jnp.float32),
            pltpu.VMEM((_S_CH,), jnp.int32),
            pltpu.VMEM_SHARED((_N_NODES, _C), jnp.float32),
        ],
    )
    return fn(msg, dst, zeros)


def _tc_final_body(agg_ref, an_ref, batch_ref, emb_ref, wout_ref, out_ref):
    agg = agg_ref[0:_N_NODES, :] + agg_ref[_N_NODES:2 * _N_NODES, :]
    an_row = an_ref[0]
    ids = lax.broadcasted_iota(jnp.int32, (_EMB_PAD, _N_NODES), 0)
    onehot_t = (ids == an_row).astype(jnp.float32)
    x = lax.dot_general(onehot_t, emb_ref[...], (((0,), (0,)), ((), ())),
                        preferred_element_type=jnp.float32)
    x = x + agg / _AVG_DEGREE
    s = jax.nn.silu(x) * wout_ref[...]
    node_e = jnp.sum(s, axis=1, keepdims=True)
    b_row = batch_ref[0]
    gids = lax.broadcasted_iota(jnp.int32, (_N_GRAPHS, _N_NODES), 0)
    gmask = (gids == b_row).astype(jnp.float32)
    e = lax.dot_general(gmask, node_e, (((1,), (0,)), ((), ())),
                        preferred_element_type=jnp.float32)
    out_ref[...] = e / _AVG_NUM_NODES


def _tc_final(agg2, an, batch, emb_pad, wout_row):
    return pl.pallas_call(
        _tc_final_body,
        out_shape=jax.ShapeDtypeStruct((_N_GRAPHS, 1), jnp.float32),
    )(agg2, an.reshape(1, _N_NODES), batch.reshape(1, _N_NODES),
      emb_pad, wout_row)


def kernel(atomic_numbers, pos, edge_index, batch, emb_table, W1, b1, W2, b2,
           W_out):
    an = atomic_numbers.astype(jnp.int32)
    src = edge_index[0].astype(jnp.int32)
    dst = edge_index[1].astype(jnp.int32)
    posx = pos[:, 0]
    posy = pos[:, 1]
    posz = pos[:, 2]
    emb_pad = jnp.concatenate(
        [emb_table, jnp.zeros((_EMB_PAD - emb_table.shape[0], _C),
                              jnp.float32)], axis=0)

    d2, ansrc = _sc_geom(posx, posy, posz, src, dst, an)
    msg = _tc_edge(d2, ansrc, emb_pad, W1, b1, W2, b2)
    zeros = jnp.zeros((_N_NODES, _C), jnp.float32)
    agg2 = _sc_scatter(msg, dst, zeros)
    energy = _tc_final(agg2.reshape(2 * _N_NODES, _C), an,
                       batch.astype(jnp.int32), emb_pad,
                       W_out.reshape(1, _C))
    return energy.reshape(_N_GRAPHS)
